# baseline (speedup 1.0000x reference)
import functools

import jax
import jax.numpy as jnp
from jax import lax
from jax.experimental import pallas as pl
from jax.experimental.pallas import tpu as pltpu
from jax.experimental.pallas import tpu_sc as plsc

V_SIZE = 1_000_000
E = 32
L_SEQ = 200
B_BATCH = 4096
B_TOTAL = B_BATCH * L_SEQ

NUM_CORES = 2
NUM_SUBCORES = 16
NW = NUM_CORES * NUM_SUBCORES

A_NBLK = V_SIZE // 512
A_TAIL0 = A_NBLK * 512

_mesh = plsc.VectorSubcoreMesh(core_axis_name="c", subcore_axis_name="s")


def _iota32():
    return jax.lax.broadcasted_iota(jnp.int32, (16,), 0) * 32


def _transpose512(src, dst, feat_major_to_token_major):
    iota32 = _iota32()
    for e in range(E):
        for tc in range(32):
            fm_off = e * 512 + 16 * tc
            tm_idx = iota32 + (512 * tc + e)
            if feat_major_to_token_major:
                vals = src[pl.ds(fm_off, 16)]
                plsc.store_scatter(dst, [tm_idx], vals)
            else:
                vals = plsc.load_gather(src, [tm_idx])
                dst[pl.ds(fm_off, 16)] = vals


@functools.partial(
    pl.kernel,
    mesh=_mesh,
    out_type=jax.ShapeDtypeStruct((V_SIZE * E,), jnp.float32),
    scratch_types=[
        pltpu.VMEM((512 * E,), jnp.float32),
        pltpu.VMEM((512 * E,), jnp.float32),
        pltpu.VMEM((512 * E,), jnp.float32),
        pltpu.VMEM((512 * E,), jnp.float32),
        pltpu.SemaphoreType.DMA,
        pltpu.SemaphoreType.DMA,
        pltpu.SemaphoreType.DMA,
        pltpu.SemaphoreType.DMA,
    ],
    compiler_params=pltpu.CompilerParams(
        needs_layout_passes=False, disable_bounds_checks=True),
)
def _reformat_table(vt_hbm, tail_hbm, out_hbm, vf0, vf1, ov0, ov1,
                    gi0, gi1, wo0, wo1):
    v2f = (vf0, vf1)
    out_v = (ov0, ov1)
    gsem = (gi0, gi1)
    wsem = (wo0, wo1)
    wid = lax.axis_index("s") * NUM_CORES + lax.axis_index("c")

    def blk_of(k):
        return k * NW + wid

    def fire_in(k, b):
        blk = blk_of(k)
        for e in range(E):
            pltpu.async_copy(
                vt_hbm.at[0, e, pl.ds(blk * 512, 512)],
                v2f[b].at[pl.ds(e * 512, 512)], gsem[b])

    @pl.when(blk_of(0) < A_NBLK)
    def _():
        fire_in(0, 0)

    @pl.loop(0, 31)
    def _pair(p):
        for b in range(2):
            k = p * 2 + b
            blk = blk_of(k)

            @pl.when(blk < A_NBLK)
            def _():
                pltpu.make_async_copy(
                    out_hbm.at[pl.ds(0, 512 * E)], v2f[b], gsem[b]).wait()

                @pl.when(blk_of(k + 1) < A_NBLK)
                def _():
                    fire_in(k + 1, 1 - b)

                @pl.when(k >= 2)
                def _():
                    pltpu.make_async_copy(
                        out_v[b], out_hbm.at[pl.ds(0, 512 * E)],
                        wsem[b]).wait()

                _transpose512(v2f[b], out_v[b], True)
                pltpu.async_copy(
                    out_v[b], out_hbm.at[pl.ds(blk * 512 * E, 512 * E)],
                    wsem[b])

    for b in range(2):
        pltpu.make_async_copy(
            out_v[b], out_hbm.at[pl.ds(0, 512 * E)], wsem[b]).wait()

    @pl.when(wid == NW - 1)
    def _():
        pltpu.sync_copy(tail_hbm, v2f[0].at[pl.ds(0, 2048)])
        pltpu.sync_copy(v2f[0].at[pl.ds(0, 2048)],
                        out_hbm.at[pl.ds(A_TAIL0 * E, 2048)])


BPW = B_TOTAL // NW
CHUNK = 640
NBUF = 4
NCHUNK = BPW // CHUNK
ROUNDS = NCHUNK // NBUF


@functools.partial(
    pl.kernel,
    mesh=_mesh,
    out_type=jax.ShapeDtypeStruct((B_TOTAL, E), jnp.float32),
    scratch_types=[
        pltpu.VMEM((BPW,), jnp.int32),
        pltpu.VMEM((NBUF, CHUNK, E), jnp.float32),
        pltpu.SemaphoreType.DMA,
        pltpu.SemaphoreType.DMA,
        pltpu.SemaphoreType.DMA,
        pltpu.SemaphoreType.DMA,
        pltpu.SemaphoreType.DMA,
        pltpu.SemaphoreType.DMA,
        pltpu.SemaphoreType.DMA,
        pltpu.SemaphoreType.DMA,
    ],
    compiler_params=pltpu.CompilerParams(use_tc_tiling_on_sc=False),
)
def _sc_gather(idx_hbm, table_hbm, out_hbm, idx_v, rows_v,
               g0, g1, g2, g3, w0, w1, w2, w3):
    gsem = (g0, g1, g2, g3)
    wsem = (w0, w1, w2, w3)
    wid = lax.axis_index("s") * NUM_CORES + lax.axis_index("c")
    base = wid * BPW

    pltpu.sync_copy(idx_hbm.at[pl.ds(base, BPW)], idx_v)

    def start_gather(chunk, buf):
        idx_slice = idx_v.at[pl.ds(chunk * CHUNK, CHUNK)]
        pltpu.async_copy(table_hbm.at[idx_slice], rows_v.at[buf], gsem[buf])

    for b in range(NBUF - 1):
        start_gather(b, b)

    @pl.loop(0, ROUNDS)
    def _round(r):
        for b in range(NBUF):
            c = r * NBUF + b
            bg = (b + NBUF - 1) % NBUF
            if b == 0:
                @pl.when(r > 0)
                def _():
                    pltpu.make_async_copy(
                        rows_v.at[bg], out_hbm.at[pl.ds(0, CHUNK)],
                        wsem[bg]).wait()
                start_gather(c + NBUF - 1, bg)
            else:
                pltpu.make_async_copy(
                    rows_v.at[bg], out_hbm.at[pl.ds(0, CHUNK)],
                    wsem[bg]).wait()

                @pl.when(r < ROUNDS - 1)
                def _():
                    start_gather(c + NBUF - 1, bg)
            pltpu.make_async_copy(
                table_hbm.at[idx_v.at[pl.ds(0, CHUNK)]], rows_v.at[b],
                gsem[b]).wait()
            pltpu.async_copy(
                rows_v.at[b], out_hbm.at[pl.ds(base + c * CHUNK, CHUNK)],
                wsem[b])

    pltpu.make_async_copy(
        rows_v.at[NBUF - 1], out_hbm.at[pl.ds(0, CHUNK)],
        wsem[NBUF - 1]).wait()


C_UNITS = B_TOTAL // 512
C_UPW = C_UNITS // NW


@functools.partial(
    pl.kernel,
    mesh=_mesh,
    out_type=jax.ShapeDtypeStruct((L_SEQ, 1, E, B_BATCH), jnp.float32),
    scratch_types=[
        pltpu.VMEM((512 * E,), jnp.float32),
        pltpu.VMEM((512 * E,), jnp.float32),
        pltpu.VMEM((512 * E,), jnp.float32),
        pltpu.VMEM((512 * E,), jnp.float32),
        pltpu.SemaphoreType.DMA,
        pltpu.SemaphoreType.DMA,
        pltpu.SemaphoreType.DMA,
        pltpu.SemaphoreType.DMA,
    ],
    compiler_params=pltpu.CompilerParams(
        needs_layout_passes=False, disable_bounds_checks=True),
)
def _reformat_out(rows_hbm, out_hbm, vt0, vt1, of0, of1, gi0, gi1, wo0, wo1):
    v2t = (vt0, vt1)
    out_vf = (of0, of1)
    gsem = (gi0, gi1)
    wsem = (wo0, wo1)
    wid = lax.axis_index("s") * NUM_CORES + lax.axis_index("c")

    def fire_in(k, b):
        u = wid * C_UPW + k
        pltpu.async_copy(rows_hbm.at[pl.ds(u * 512 * E, 512 * E)],
                         v2t[b], gsem[b])

    def fire_out(k, b):
        u = wid * C_UPW + k
        l = u // 8
        jb = lax.rem(u, 8)
        for e in range(E):
            pltpu.async_copy(
                out_vf[b].at[pl.ds(e * 512, 512)],
                out_hbm.at[l, 0, e, pl.ds(jb * 512, 512)], wsem[b])

    fire_in(0, 0)

    @pl.loop(0, 25)
    def _pair(p):
        for b in range(2):
            k = p * 2 + b
            pltpu.make_async_copy(
                rows_hbm.at[pl.ds(0, 512 * E)], v2t[b], gsem[b]).wait()

            @pl.when(k + 1 < C_UPW)
            def _():
                fire_in(k + 1, 1 - b)

            @pl.when(k >= 2)
            def _():
                for _ in range(E):
                    pltpu.make_async_copy(
                        out_vf[b].at[pl.ds(0, 512)],
                        out_hbm.at[0, 0, 0, pl.ds(0, 512)], wsem[b]).wait()

            _transpose512(v2t[b], out_vf[b], False)
            fire_out(k, b)

    for b in range(2):
        for _ in range(E):
            pltpu.make_async_copy(
                out_vf[b].at[pl.ds(0, 512)],
                out_hbm.at[0, 0, 0, pl.ds(0, 512)], wsem[b]).wait()


def kernel(x, vocab):
    vt = jnp.transpose(vocab, (1, 2, 0))
    tail = vocab[A_TAIL0:, 0, :].reshape(-1)
    table_flat = _reformat_table(vt, tail)
    table = table_flat.reshape(V_SIZE, E)

    idx = jnp.transpose(x).reshape(-1).astype(jnp.int32)
    rows = _sc_gather(idx, table)
    rows_flat = rows.reshape(-1)
    out = _reformat_out(rows_flat)
    return jnp.transpose(out, (3, 0, 1, 2))


# speedup vs baseline: 1.4806x; 1.4806x over previous
import functools

import jax
import jax.numpy as jnp
from jax import lax
from jax.experimental import pallas as pl
from jax.experimental.pallas import tpu as pltpu
from jax.experimental.pallas import tpu_sc as plsc

V_SIZE = 1_000_000
E = 32
B_TOTAL = 4096 * 200

NUM_CORES = 2
NUM_SUBCORES = 16
NW = NUM_CORES * NUM_SUBCORES
BPW = B_TOTAL // NW
CHUNK = 1600
NCHUNK = BPW // CHUNK

_mesh = plsc.VectorSubcoreMesh(core_axis_name="c", subcore_axis_name="s")


@functools.partial(
    pl.kernel,
    mesh=_mesh,
    out_type=jax.ShapeDtypeStruct((B_TOTAL, E), jnp.float32),
    scratch_types=[
        pltpu.VMEM((BPW,), jnp.int32),
        pltpu.VMEM((CHUNK, E), jnp.float32),
        pltpu.SemaphoreType.DMA,
    ],
    compiler_params=pltpu.CompilerParams(use_tc_tiling_on_sc=False),
)
def _sc_gather(idx_hbm, table_hbm, out_hbm, idx_v, rows_v, sem):
    wid = lax.axis_index("s") * NUM_CORES + lax.axis_index("c")
    base = wid * BPW

    pltpu.sync_copy(idx_hbm.at[pl.ds(base, BPW)], idx_v)

    def body(i, carry):
        idx_slice = idx_v.at[pl.ds(i * CHUNK, CHUNK)]
        pltpu.async_copy(table_hbm.at[idx_slice], rows_v, sem).wait()
        pltpu.sync_copy(rows_v, out_hbm.at[pl.ds(base + i * CHUNK, CHUNK)])
        return carry

    lax.fori_loop(0, NCHUNK, body, 0)


def kernel(x, vocab):
    idx = x.reshape(-1).astype(jnp.int32)
    table = vocab.reshape(V_SIZE, E)
    out = _sc_gather(idx, table)
    return out.reshape(x.shape[0], x.shape[1], 1, E)
